# baseline (speedup 1.0000x reference)
import functools

import jax
import jax.numpy as jnp
from jax.experimental import pallas as pl
from jax.experimental.pallas import tpu as pltpu

N = 16384
D_IN = 64
D_OUT = 64
BM = 256


def _support_kernel(x_ref, w_ref, out_ref):
    out_ref[...] = jnp.dot(x_ref[...], w_ref[...],
                           preferred_element_type=jnp.float32
                           ).astype(jnp.bfloat16)


def _spmm_kernel(adj_lo_ref, adj_hi_ref, support_ref, bias_ref, out_ref):
    a_lo = adj_lo_ref[...].astype(jnp.bfloat16)
    a_hi = adj_hi_ref[...].astype(jnp.bfloat16)
    acc = jnp.dot(a_lo, support_ref[: N // 2],
                  preferred_element_type=jnp.float32)
    acc += jnp.dot(a_hi, support_ref[N // 2:],
                   preferred_element_type=jnp.float32)
    out_ref[...] = acc + bias_ref[...]


@jax.jit
def kernel(x, adj, weight, bias):
    support = pl.pallas_call(
        _support_kernel,
        out_shape=jax.ShapeDtypeStruct((N, D_OUT), jnp.bfloat16),
    )(x, weight)

    bias2d = bias.reshape(1, D_OUT)
    out = pl.pallas_call(
        _spmm_kernel,
        grid=(N // BM,),
        in_specs=[
            pl.BlockSpec((BM, N // 2), lambda i: (i, 0)),
            pl.BlockSpec((BM, N // 2), lambda i: (i, 1)),
            pl.BlockSpec((N, D_OUT), lambda i: (0, 0)),
            pl.BlockSpec((1, D_OUT), lambda i: (0, 0)),
        ],
        out_specs=pl.BlockSpec((BM, D_OUT), lambda i: (i, 0)),
        out_shape=jax.ShapeDtypeStruct((N, D_OUT), jnp.float32),
        compiler_params=pltpu.CompilerParams(
            dimension_semantics=("parallel",),
        ),
    )(adj, adj, support, bias2d)
    return out


# speedup vs baseline: 1.0187x; 1.0187x over previous
import jax
import jax.numpy as jnp
from jax.experimental import pallas as pl
from jax.experimental.pallas import tpu as pltpu

N = 16384
D_IN = 64
D_OUT = 64
BM = 256


def _fused_kernel(x_ref, w_ref, bias_ref, adj_ref, out_ref, s_ref):
    @pl.when(pl.program_id(0) == 0)
    def _():
        s_ref[...] = jnp.dot(x_ref[...], w_ref[...],
                             preferred_element_type=jnp.float32
                             ).astype(jnp.bfloat16)

    a = adj_ref[...].astype(jnp.bfloat16)
    out_ref[...] = jnp.dot(a, s_ref[...],
                           preferred_element_type=jnp.float32) + bias_ref[...]


@jax.jit
def kernel(x, adj, weight, bias):
    bias2d = bias.reshape(1, D_OUT)
    out = pl.pallas_call(
        _fused_kernel,
        grid=(N // BM,),
        in_specs=[
            pl.BlockSpec((N, D_IN), lambda i: (0, 0)),
            pl.BlockSpec((D_IN, D_OUT), lambda i: (0, 0)),
            pl.BlockSpec((1, D_OUT), lambda i: (0, 0)),
            pl.BlockSpec((BM, N), lambda i: (i, 0)),
        ],
        out_specs=pl.BlockSpec((BM, D_OUT), lambda i: (i, 0)),
        out_shape=jax.ShapeDtypeStruct((N, D_OUT), jnp.float32),
        scratch_shapes=[pltpu.VMEM((N, D_OUT), jnp.bfloat16)],
        compiler_params=pltpu.CompilerParams(
            dimension_semantics=("arbitrary",),
        ),
    )(x, weight, bias2d, adj)
    return out
